# baseline (speedup 1.0000x reference)
import jax
import jax.numpy as jnp
from jax.experimental import pallas as pl
from jax.experimental.pallas import tpu as pltpu

_N = 16384
_D = 512
_CH = 1024
_NCH = _N // _CH
_NBUF = 8


def _concat_body(idx_ref, x_hbm, w_hbm, out_hbm, *scratch):
    bufs = scratch[:_NBUF]
    row_buf = scratch[_NBUF]
    sins = scratch[_NBUF + 1:2 * _NBUF + 1]
    souts = scratch[2 * _NBUF + 1:3 * _NBUF + 1]
    sem_row_in, sem_row_out = scratch[3 * _NBUF + 1:]

    p = idx_ref[0]
    o = idx_ref[1]
    row_in = pltpu.make_async_copy(w_hbm.at[p, o], row_buf, sem_row_in)
    row_in.start()

    in_cps = [None] * _NCH
    out_cps = [None] * _NCH

    def start_in(k):
        b = k % _NBUF
        cp = pltpu.make_async_copy(
            x_hbm.at[pl.ds(k * _CH, _CH)], bufs[b], sins[b]
        )
        cp.start()
        in_cps[k] = cp

    def start_out(k):
        b = k % _NBUF
        cp = pltpu.make_async_copy(
            bufs[b], out_hbm.at[pl.ds(k * _CH, _CH)], souts[b]
        )
        cp.start()
        out_cps[k] = cp

    for k in range(_NBUF - 1):
        start_in(k)
    row_in.wait()
    row_out = pltpu.make_async_copy(
        row_buf, out_hbm.at[pl.ds(_N, 1)], sem_row_out
    )
    row_out.start()
    for k in range(_NCH):
        in_cps[k].wait()
        start_out(k)
        if k + _NBUF - 1 < _NCH:
            if k >= 1:
                out_cps[k - 1].wait()
            start_in(k + _NBUF - 1)
    for k in range(max(0, _NCH - _NBUF), _NCH):
        out_cps[k].wait()
    row_out.wait()


def kernel(input_features, part_cls, obj_cls, W):
    idx = jnp.stack(
        [jnp.asarray(part_cls, jnp.int32), jnp.asarray(obj_cls, jnp.int32)]
    )
    scratch_shapes = (
        [pltpu.VMEM((_CH, _D), jnp.float32)] * _NBUF
        + [pltpu.VMEM((1, _D), jnp.float32)]
        + [pltpu.SemaphoreType.DMA] * (2 * _NBUF + 2)
    )
    return pl.pallas_call(
        _concat_body,
        grid=(),
        in_specs=[
            pl.BlockSpec(memory_space=pltpu.SMEM),
            pl.BlockSpec(memory_space=pl.ANY),
            pl.BlockSpec(memory_space=pl.ANY),
        ],
        out_specs=pl.BlockSpec(memory_space=pl.ANY),
        out_shape=jax.ShapeDtypeStruct((_N + 1, _D), jnp.float32),
        scratch_shapes=scratch_shapes,
    )(idx, input_features, W)


# speedup vs baseline: 1.0178x; 1.0178x over previous
import jax
import jax.numpy as jnp
from jax.experimental import pallas as pl
from jax.experimental.pallas import tpu as pltpu

_N = 16384
_D = 512
_BLK = 4096
_GRID = _N // _BLK + 1


def _concat_body(idx_ref, x_ref, w_ref, o_ref):
    i = pl.program_id(0)

    @pl.when(i < _GRID - 1)
    def _copy():
        o_ref[...] = x_ref[...]

    @pl.when(i == _GRID - 1)
    def _tail():
        o_ref[...] = jnp.broadcast_to(w_ref[0, 0], (_BLK, _D))


def kernel(input_features, part_cls, obj_cls, W):
    idx = jnp.stack(
        [jnp.asarray(part_cls, jnp.int32), jnp.asarray(obj_cls, jnp.int32)]
    )
    grid_spec = pltpu.PrefetchScalarGridSpec(
        num_scalar_prefetch=1,
        grid=(_GRID,),
        in_specs=[
            pl.BlockSpec(
                (_BLK, _D), lambda i, idx: (jnp.minimum(i, _N // _BLK - 1), 0)
            ),
            pl.BlockSpec((1, 1, 1, _D), lambda i, idx: (idx[0], idx[1], 0, 0)),
        ],
        out_specs=pl.BlockSpec((_BLK, _D), lambda i, idx: (i, 0)),
    )
    return pl.pallas_call(
        _concat_body,
        grid_spec=grid_spec,
        out_shape=jax.ShapeDtypeStruct((_N + 1, _D), jnp.float32),
    )(idx, input_features, W)
